# Initial kernel scaffold; baseline (speedup 1.0000x reference)
import jax
import jax.numpy as jnp
from jax.experimental import pallas as pl


def kernel(inputs, edge_index, edge_type, training, embed, type_w, type_b, gru_wx, gru_wh, gru_b):
    raise NotImplementedError("write your pallas kernel here")


# speedup vs baseline: 4.2152x; 4.2152x over previous
import functools

import jax
import jax.numpy as jnp
from jax import lax
from jax.experimental import pallas as pl
from jax.experimental.pallas import tpu as pltpu
from jax.experimental.pallas import tpu_sc as plsc

H = 128
H3 = 3 * H
N_NODES = 10000
N_EDGES = 160000
NUM_TYPES = 3
STEPS = [2, 1]

NC = 2
NS = 16
NW = NC * NS

N_PAD = 10240
E_PAD = 163840
EPW = E_PAD // NW
CHUNK = 128
NCHUNK = EPW // CHUNK
RPT = N_PAD // NS
ZROWS = 128

_mesh = plsc.VectorSubcoreMesh(core_axis_name="c", subcore_axis_name="s")


@functools.partial(
    pl.kernel,
    out_type=jax.ShapeDtypeStruct((N_PAD, H), jnp.float32),
    mesh=_mesh,
    scratch_types=[
        pltpu.VMEM((80,), jnp.int32),
        pltpu.VMEM((80, H), jnp.float32),
        pltpu.SemaphoreType.DMA,
    ],
)
def _embed_lookup(emb, ids, out, idx_v, rows_v, sem):
    wid = lax.axis_index("c") * NS + lax.axis_index("s")

    def body(k, _):
        base = wid * 320 + k * 80
        pltpu.sync_copy(ids.at[pl.ds(base, 80)], idx_v)
        pltpu.async_copy(emb.at[idx_v], rows_v, sem).wait()
        pltpu.sync_copy(rows_v, out.at[pl.ds(base, 80)])
        return 0

    lax.fori_loop(0, 4, body, 0)


@functools.partial(
    pl.kernel,
    out_type=jax.ShapeDtypeStruct((NC, N_PAD, H), jnp.float32),
    mesh=_mesh,
    scratch_types=[
        pltpu.VMEM((CHUNK,), jnp.int32),
        pltpu.VMEM((CHUNK,), jnp.int32),
        pltpu.VMEM((CHUNK,), jnp.int32),
        pltpu.VMEM((CHUNK,), jnp.int32),
        pltpu.VMEM((CHUNK, H), jnp.float32),
        pltpu.VMEM_SHARED((N_PAD, H), jnp.float32),
        pltpu.SemaphoreType.DMA,
    ],
)
def _messages(table, zeros, srca, typa, tgta, out,
              src_v, typ_v, tgt_v, gidx_v, rows_v, acc, sem):
    c = lax.axis_index("c")
    s = lax.axis_index("s")
    wid = c * NS + s

    def zbody(k, _):
        pltpu.sync_copy(zeros, acc.at[pl.ds(s * RPT + k * ZROWS, ZROWS)])
        return 0

    lax.fori_loop(0, RPT // ZROWS, zbody, 0)
    plsc.subcore_barrier()

    def ebody(k, _):
        base = wid * EPW + k * CHUNK
        pltpu.sync_copy(srca.at[pl.ds(base, CHUNK)], src_v)
        pltpu.sync_copy(typa.at[pl.ds(base, CHUNK)], typ_v)
        pltpu.sync_copy(tgta.at[pl.ds(base, CHUNK)], tgt_v)
        for i in range(CHUNK // 16):
            sl = pl.ds(i * 16, 16)
            gidx_v[sl] = src_v[sl] * 3 + typ_v[sl]
        pltpu.async_copy(table.at[gidx_v], rows_v, sem).wait()
        pltpu.sync_copy(rows_v, acc.at[tgt_v], add=True)
        return 0

    lax.fori_loop(0, NCHUNK, ebody, 0)
    plsc.subcore_barrier()

    pltpu.sync_copy(acc.at[pl.ds(s * RPT, RPT)], out.at[c, pl.ds(s * RPT, RPT)])


_BN = 1024


def _transform_body(s_ref, w_ref, b_ref, o_ref):
    o_ref[...] = (
        jnp.dot(s_ref[...], w_ref[...], preferred_element_type=jnp.float32)
        + b_ref[...]
    )


def _transform(states, wcat, bcat):
    return pl.pallas_call(
        _transform_body,
        grid=(N_PAD // _BN,),
        in_specs=[
            pl.BlockSpec((_BN, H), lambda i: (i, 0)),
            pl.BlockSpec((H, H3), lambda i: (0, 0)),
            pl.BlockSpec((1, H3), lambda i: (0, 0)),
        ],
        out_specs=pl.BlockSpec((_BN, H3), lambda i: (i, 0)),
        out_shape=jax.ShapeDtypeStruct((N_PAD, H3), jnp.float32),
    )(states, wcat, bcat)


def _gru_body(p_ref, s_ref, wx_ref, wh_ref, b_ref, o_ref):
    m = p_ref[0] + p_ref[1]
    h = s_ref[...]
    gx = jnp.dot(m, wx_ref[...], preferred_element_type=jnp.float32) + b_ref[...]
    gh = jnp.dot(h, wh_ref[...], preferred_element_type=jnp.float32)
    z = jax.nn.sigmoid(gx[:, :H] + gh[:, :H])
    r = jax.nn.sigmoid(gx[:, H:2 * H] + gh[:, H:2 * H])
    hcand = jnp.tanh(gx[:, 2 * H:] + r * gh[:, 2 * H:])
    o_ref[...] = z * h + (1.0 - z) * hcand


def _gru(partials, states, wx, wh, b):
    return pl.pallas_call(
        _gru_body,
        grid=(N_PAD // _BN,),
        in_specs=[
            pl.BlockSpec((NC, _BN, H), lambda i: (0, i, 0)),
            pl.BlockSpec((_BN, H), lambda i: (i, 0)),
            pl.BlockSpec((H, H3), lambda i: (0, 0)),
            pl.BlockSpec((H, H3), lambda i: (0, 0)),
            pl.BlockSpec((1, H3), lambda i: (0, 0)),
        ],
        out_specs=pl.BlockSpec((_BN, H), lambda i: (i, 0)),
        out_shape=jax.ShapeDtypeStruct((N_PAD, H), jnp.float32),
    )(partials, states, wx, wh, b)


def kernel(inputs, edge_index, edge_type, training, embed, type_w, type_b,
           gru_wx, gru_wh, gru_b):
    del training
    src = edge_index[0]
    tgt = edge_index[1]
    epad = E_PAD - N_EDGES
    srcp = jnp.concatenate([src, jnp.zeros((epad,), jnp.int32)])
    typp = jnp.concatenate([edge_type, jnp.zeros((epad,), jnp.int32)])
    tgtp = jnp.concatenate([tgt, jnp.full((epad,), N_NODES, jnp.int32)])
    idsp = jnp.concatenate([inputs, jnp.zeros((N_PAD - N_NODES,), jnp.int32)])
    zeros = jnp.zeros((ZROWS, H), jnp.float32)

    states = _embed_lookup(embed, idsp)

    for layer in range(len(STEPS)):
        wcat = type_w[layer].transpose(1, 0, 2).reshape(H, H3)
        bcat = type_b[layer].reshape(1, H3)
        wx = gru_wx[layer]
        wh = gru_wh[layer]
        b = gru_b[layer].reshape(1, H3)
        for _ in range(STEPS[layer]):
            tall = _transform(states, wcat, bcat)
            table = tall.reshape(NUM_TYPES * N_PAD, H)
            partials = _messages(table, zeros, srcp, typp, tgtp)
            states = _gru(partials, states, wx, wh, b)

    return states[:N_NODES]
